# baseline (speedup 1.0000x reference)
import functools

import jax
import jax.numpy as jnp
from jax import lax
from jax.experimental import pallas as pl
from jax.experimental.pallas import tpu as pltpu
from jax.experimental.pallas import tpu_sc as plsc

_NC = 2
_NS = 16
_C = 128
_EPS = 1e-7
_BN_INV = 1.0 / (1.0 + 1e-5) ** 0.5


def _tables_body(x_ref, p_ref, pm_ref):
    m = jnp.maximum(x_ref[...], 0.0) + _EPS
    p = jnp.exp(m)
    p_ref[...] = p
    pm_ref[...] = p * m


def _make_tables(x, block_rows):
    n, d = x.shape
    return pl.pallas_call(
        _tables_body,
        grid=(n // block_rows,),
        in_specs=[pl.BlockSpec((block_rows, d), lambda i: (i, 0))],
        out_specs=[pl.BlockSpec((block_rows, d), lambda i: (i, 0)),
                   pl.BlockSpec((block_rows, d), lambda i: (i, 0))],
        out_shape=[jax.ShapeDtypeStruct((n, d), jnp.float32),
                   jax.ShapeDtypeStruct((n, d), jnp.float32)],
    )(x)


_G = 40


def _sc_edge_phase(ei, p_tab, pm_tab, zeros, n, k_chunks, s_rows, d):
    mesh = plsc.VectorSubcoreMesh(core_axis_name="c", subcore_axis_name="s")
    n_groups = k_chunks // _G

    @functools.partial(
        pl.kernel,
        out_type=jax.ShapeDtypeStruct((_NC, n, d), jnp.float32),
        mesh=mesh,
        scratch_types=[
            pltpu.VMEM((_G, _C), jnp.int32),
            pltpu.VMEM((_G, _C), jnp.int32),
            pltpu.VMEM((_C, d), jnp.float32),
            pltpu.VMEM((_C, d), jnp.float32),
            pltpu.VMEM_SHARED((s_rows, d), jnp.float32),
            pltpu.SemaphoreType.DMA,
            pltpu.SemaphoreType.DMA,
            pltpu.SemaphoreType.DMA,
            pltpu.SemaphoreType.DMA,
        ],
    )
    def edge_kernel(ei_hbm, p_hbm, pm_hbm, z_hbm, out_hbm,
                    src_v, dst_v, rows0, rows1, s_sh,
                    gsem0, gsem1, ssem0, ssem1):
        cid = lax.axis_index("c")
        sid = lax.axis_index("s")

        @pl.when(sid == 0)
        def _():
            pltpu.sync_copy(z_hbm, s_sh)

        plsc.subcore_barrier()

        def run(tab):
            def gwait(buf, sem):
                pltpu.make_async_copy(tab.at[pl.ds(0, _C)], buf, sem).wait()

            def swait(sem):
                pltpu.make_async_copy(
                    tab.at[pl.ds(0, _C)], s_sh.at[pl.ds(0, _C)], sem).wait()

            def outer(g, carry):
                pltpu.sync_copy(ei_hbm.at[0, sid, pl.ds(g * _G, _G)], src_v)
                pltpu.sync_copy(ei_hbm.at[1, sid, pl.ds(g * _G, _G)], dst_v)
                pltpu.async_copy(tab.at[src_v.at[0]], rows0, gsem0)
                pltpu.async_copy(tab.at[src_v.at[1]], rows1, gsem1)

                def pair(t, c2):
                    a = 2 * t
                    gwait(rows0, gsem0)
                    pltpu.async_copy(rows0, s_sh.at[dst_v.at[a]], ssem0,
                                     add=True)
                    gwait(rows1, gsem1)
                    swait(ssem0)
                    pltpu.async_copy(rows1, s_sh.at[dst_v.at[a + 1]], ssem1,
                                     add=True)

                    @pl.when(t < _G // 2 - 1)
                    def _():
                        pltpu.async_copy(tab.at[src_v.at[a + 2]], rows0, gsem0)

                    swait(ssem1)

                    @pl.when(t < _G // 2 - 1)
                    def _():
                        pltpu.async_copy(tab.at[src_v.at[a + 3]], rows1, gsem1)

                    return c2

                lax.fori_loop(0, _G // 2, pair, 0)
                return carry

            lax.fori_loop(0, n_groups, outer, 0)

        @pl.when(cid == 0)
        def _():
            run(p_hbm)

        @pl.when(cid == 1)
        def _():
            run(pm_hbm)

        plsc.subcore_barrier()

        @pl.when(sid == 0)
        def _():
            pltpu.sync_copy(s_sh.at[pl.ds(0, n)], out_hbm.at[cid])

    return edge_kernel(ei, p_tab, pm_tab, zeros)


def _dense_body(s_ref, x_ref, w1_ref, b1_ref, g1_ref, be1_ref,
                w2_ref, b2_ref, g2_ref, be2_ref, o_ref):
    s0 = s_ref[0]
    s1 = s_ref[1]
    aggr = jnp.where(s0 > 0.0, s1 / s0, 0.0)
    out = aggr + x_ref[...]
    h = jnp.dot(out, w1_ref[...], preferred_element_type=jnp.float32) + b1_ref[...]
    h = h * (g1_ref[...] * _BN_INV) + be1_ref[...]
    h = jnp.maximum(h, 0.0)
    y = jnp.dot(h, w2_ref[...], preferred_element_type=jnp.float32) + b2_ref[...]
    mu = jnp.mean(y, axis=-1, keepdims=True)
    var = jnp.mean((y - mu) ** 2, axis=-1, keepdims=True)
    y = (y - mu) * lax.rsqrt(var + 1e-5) * g2_ref[...] + be2_ref[...]
    o_ref[...] = jnp.maximum(y, 0.0)


def _dense_phase(s, x, W1, b1, bn_gamma, bn_beta, W2, b2, ln_gamma, ln_beta,
                 block_rows):
    n, d = x.shape
    full = lambda shape: pl.BlockSpec(shape, lambda i: tuple(0 for _ in shape))
    return pl.pallas_call(
        _dense_body,
        grid=(n // block_rows,),
        in_specs=[
            pl.BlockSpec((2, block_rows, d), lambda i: (0, i, 0)),
            pl.BlockSpec((block_rows, d), lambda i: (i, 0)),
            full((d, 2 * d)),
            full((1, 2 * d)),
            full((1, 2 * d)),
            full((1, 2 * d)),
            full((2 * d, d)),
            full((1, d)),
            full((1, d)),
            full((1, d)),
        ],
        out_specs=pl.BlockSpec((block_rows, d), lambda i: (i, 0)),
        out_shape=jax.ShapeDtypeStruct((n, d), jnp.float32),
    )(s, x, W1, b1.reshape(1, -1), bn_gamma.reshape(1, -1),
      bn_beta.reshape(1, -1), W2, b2.reshape(1, -1),
      ln_gamma.reshape(1, -1), ln_beta.reshape(1, -1))


def kernel(x, edge_index, W1, b1, bn_gamma, bn_beta, W2, b2, ln_gamma, ln_beta):
    n, d = x.shape
    e = edge_index.shape[1]

    p_tab, pm_tab = _make_tables(x, 2000)

    k_chunks = -(-e // (_NS * _C * _G)) * _G
    e_pad = _NS * k_chunks * _C
    pad = e_pad - e
    src_p = jnp.concatenate([edge_index[0], jnp.zeros((pad,), jnp.int32)])
    dst_p = jnp.concatenate([edge_index[1], jnp.full((pad,), n, jnp.int32)])
    ei = jnp.stack([src_p, dst_p]).reshape(2, _NS, k_chunks, _C)

    s_rows = n + 8
    zeros = jnp.zeros((s_rows, d), jnp.float32)

    s = _sc_edge_phase(ei, p_tab, pm_tab, zeros, n, k_chunks, s_rows, d)

    return _dense_phase(s, x, W1, b1, bn_gamma, bn_beta, W2, b2,
                        ln_gamma, ln_beta, 2000)


# speedup vs baseline: 1.0259x; 1.0259x over previous
import functools

import jax
import jax.numpy as jnp
from jax import lax
from jax.experimental import pallas as pl
from jax.experimental.pallas import tpu as pltpu
from jax.experimental.pallas import tpu_sc as plsc

_NC = 2
_NS = 16
_C = 128
_EPS = 1e-7
_BN_INV = 1.0 / (1.0 + 1e-5) ** 0.5


def _tables_body(x_ref, p_ref, pm_ref):
    m = jnp.maximum(x_ref[...], 0.0) + _EPS
    p = jnp.exp(m)
    p_ref[...] = p
    pm_ref[...] = p * m


def _make_tables(x, block_rows):
    n, d = x.shape
    return pl.pallas_call(
        _tables_body,
        grid=(n // block_rows,),
        in_specs=[pl.BlockSpec((block_rows, d), lambda i: (i, 0))],
        out_specs=[pl.BlockSpec((block_rows, d), lambda i: (i, 0)),
                   pl.BlockSpec((block_rows, d), lambda i: (i, 0))],
        out_shape=[jax.ShapeDtypeStruct((n, d), jnp.float32),
                   jax.ShapeDtypeStruct((n, d), jnp.float32)],
    )(x)


_G = 40


def _sc_edge_phase(ei, p_tab, pm_tab, zeros, n, k_chunks, s_rows, d):
    mesh = plsc.VectorSubcoreMesh(core_axis_name="c", subcore_axis_name="s")
    n_groups = k_chunks // _G

    @functools.partial(
        pl.kernel,
        out_type=jax.ShapeDtypeStruct((_NC, n, d), jnp.float32),
        mesh=mesh,
        scratch_types=[
            pltpu.VMEM((_G, _C), jnp.int32),
            pltpu.VMEM((_G, _C), jnp.int32),
            pltpu.VMEM((_C, d), jnp.float32),
            pltpu.VMEM((_C, d), jnp.float32),
            pltpu.VMEM_SHARED((s_rows, d), jnp.float32),
            pltpu.SemaphoreType.DMA,
            pltpu.SemaphoreType.DMA,
        ],
    )
    def edge_kernel(ei_hbm, p_hbm, pm_hbm, z_hbm, out_hbm,
                    src_v, dst_v, rows0, rows1, s_sh,
                    gsem0, gsem1):
        cid = lax.axis_index("c")
        sid = lax.axis_index("s")

        @pl.when(sid == 0)
        def _():
            pltpu.sync_copy(z_hbm, s_sh)

        plsc.subcore_barrier()

        def run(tab):
            def gwait(buf, sem):
                pltpu.make_async_copy(tab.at[pl.ds(0, _C)], buf, sem).wait()

            def outer(g, carry):
                pltpu.sync_copy(ei_hbm.at[0, sid, pl.ds(g * _G, _G)], src_v)
                pltpu.sync_copy(ei_hbm.at[1, sid, pl.ds(g * _G, _G)], dst_v)
                pltpu.async_copy(tab.at[src_v.at[0]], rows0, gsem0)
                pltpu.async_copy(tab.at[src_v.at[1]], rows1, gsem1)

                def pair(t, c2):
                    a = 2 * t
                    gwait(rows0, gsem0)
                    pltpu.sync_copy(rows0, s_sh.at[dst_v.at[a]], add=True)

                    @pl.when(t < _G // 2 - 1)
                    def _():
                        pltpu.async_copy(tab.at[src_v.at[a + 2]], rows0, gsem0)

                    gwait(rows1, gsem1)
                    pltpu.sync_copy(rows1, s_sh.at[dst_v.at[a + 1]], add=True)

                    @pl.when(t < _G // 2 - 1)
                    def _():
                        pltpu.async_copy(tab.at[src_v.at[a + 3]], rows1, gsem1)

                    return c2

                lax.fori_loop(0, _G // 2, pair, 0)
                return carry

            lax.fori_loop(0, n_groups, outer, 0)

        @pl.when(cid == 0)
        def _():
            run(p_hbm)

        @pl.when(cid == 1)
        def _():
            run(pm_hbm)

        plsc.subcore_barrier()

        @pl.when(sid == 0)
        def _():
            pltpu.sync_copy(s_sh.at[pl.ds(0, n)], out_hbm.at[cid])

    return edge_kernel(ei, p_tab, pm_tab, zeros)


def _dense_body(s_ref, x_ref, w1_ref, b1_ref, g1_ref, be1_ref,
                w2_ref, b2_ref, g2_ref, be2_ref, o_ref):
    s0 = s_ref[0]
    s1 = s_ref[1]
    aggr = jnp.where(s0 > 0.0, s1 / s0, 0.0)
    out = aggr + x_ref[...]
    h = jnp.dot(out, w1_ref[...], preferred_element_type=jnp.float32) + b1_ref[...]
    h = h * (g1_ref[...] * _BN_INV) + be1_ref[...]
    h = jnp.maximum(h, 0.0)
    y = jnp.dot(h, w2_ref[...], preferred_element_type=jnp.float32) + b2_ref[...]
    mu = jnp.mean(y, axis=-1, keepdims=True)
    var = jnp.mean((y - mu) ** 2, axis=-1, keepdims=True)
    y = (y - mu) * lax.rsqrt(var + 1e-5) * g2_ref[...] + be2_ref[...]
    o_ref[...] = jnp.maximum(y, 0.0)


def _dense_phase(s, x, W1, b1, bn_gamma, bn_beta, W2, b2, ln_gamma, ln_beta,
                 block_rows):
    n, d = x.shape
    full = lambda shape: pl.BlockSpec(shape, lambda i: tuple(0 for _ in shape))
    return pl.pallas_call(
        _dense_body,
        grid=(n // block_rows,),
        in_specs=[
            pl.BlockSpec((2, block_rows, d), lambda i: (0, i, 0)),
            pl.BlockSpec((block_rows, d), lambda i: (i, 0)),
            full((d, 2 * d)),
            full((1, 2 * d)),
            full((1, 2 * d)),
            full((1, 2 * d)),
            full((2 * d, d)),
            full((1, d)),
            full((1, d)),
            full((1, d)),
        ],
        out_specs=pl.BlockSpec((block_rows, d), lambda i: (i, 0)),
        out_shape=jax.ShapeDtypeStruct((n, d), jnp.float32),
    )(s, x, W1, b1.reshape(1, -1), bn_gamma.reshape(1, -1),
      bn_beta.reshape(1, -1), W2, b2.reshape(1, -1),
      ln_gamma.reshape(1, -1), ln_beta.reshape(1, -1))


def kernel(x, edge_index, W1, b1, bn_gamma, bn_beta, W2, b2, ln_gamma, ln_beta):
    n, d = x.shape
    e = edge_index.shape[1]

    p_tab, pm_tab = _make_tables(x, 2000)

    k_chunks = -(-e // (_NS * _C * _G)) * _G
    e_pad = _NS * k_chunks * _C
    pad = e_pad - e
    src_p = jnp.concatenate([edge_index[0], jnp.zeros((pad,), jnp.int32)])
    dst_p = jnp.concatenate([edge_index[1], jnp.full((pad,), n, jnp.int32)])
    ei = jnp.stack([src_p, dst_p]).reshape(2, _NS, k_chunks, _C)

    s_rows = n + 8
    zeros = jnp.zeros((s_rows, d), jnp.float32)

    s = _sc_edge_phase(ei, p_tab, pm_tab, zeros, n, k_chunks, s_rows, d)

    return _dense_phase(s, x, W1, b1, bn_gamma, bn_beta, W2, b2,
                        ln_gamma, ln_beta, 2000)
